# baseline (speedup 1.0000x reference)
import functools

import jax
import jax.numpy as jnp
from jax import lax
from jax.experimental import pallas as pl
from jax.experimental.pallas import tpu as pltpu
from jax.experimental.pallas import tpu_sc as plsc

NUM_ROWS = 100000
EMB = 32
KH = 4
HC = 73244475
HC0 = HC & 0xFFFF
HC1 = HC >> 16
M32 = 16824
M64C = 100000 - 51616

SEQ = 16384
WORDS = 26
N_TOK = SEQ * WORDS
LANES = 128
ROWS2 = N_TOK // LANES
HB = 128
NW = 32
SEQ_PER_W = SEQ // NW
SEQ_BLK = 16
NBLK = SEQ_PER_W // SEQ_BLK
TOK_BLK = SEQ_BLK * WORDS
IDX_CHUNKS = KH * TOK_BLK // LANES


def _hash_block_i64(t_ref, o_ref):
    x = t_ref[...].astype(jnp.uint32)
    outs = []
    for r in range(KH):
        lo = x + jnp.uint32(r)
        hi = jnp.zeros_like(lo)
        for _ in range(2):
            s_lo = ((lo >> 16) | (hi << 16)) ^ lo
            s_hi = (hi.astype(jnp.int32) >> 16).astype(jnp.uint32) ^ hi
            m_lo = s_lo * jnp.uint32(HC)
            x0 = s_lo & jnp.uint32(0xFFFF)
            x1 = s_lo >> 16
            m0 = x0 * jnp.uint32(HC0)
            mid = x1 * jnp.uint32(HC0) + x0 * jnp.uint32(HC1) + (m0 >> 16)
            mhi = x1 * jnp.uint32(HC1) + (mid >> 16)
            hi = s_hi * jnp.uint32(HC) + mhi
            lo = m_lo
        f_lo = ((lo >> 16) | (hi << 16)) ^ lo
        f_hi = (hi.astype(jnp.int32) >> 16).astype(jnp.uint32) ^ hi
        m = jnp.uint32(NUM_ROWS)
        p = ((f_hi % m) * jnp.uint32(M32)) % m
        q = (jnp.uint32(4) * p + f_lo % m) % m
        q = jnp.where(f_hi.astype(jnp.int32) < 0, (q + jnp.uint32(M64C)) % m, q)
        outs.append(q.astype(jnp.int32))
    o_ref[...] = jnp.stack(outs)


def _hash_block_i32(t_ref, o_ref):
    outs = []
    for r in range(KH):
        t = t_ref[...] + r
        t = ((t >> 16) ^ t) * HC
        t = ((t >> 16) ^ t) * HC
        t = (t >> 16) ^ t
        outs.append(jnp.mod(t, NUM_ROWS))
    o_ref[...] = jnp.stack(outs)


def _compute_idx(t32, exact_i64):
    body = _hash_block_i64 if exact_i64 else _hash_block_i32
    return pl.pallas_call(
        body,
        grid=(ROWS2 // HB,),
        in_specs=[pl.BlockSpec((HB, LANES), lambda i: (i, jnp.int32(0)))],
        out_specs=pl.BlockSpec(
            (KH, HB, LANES), lambda i: (jnp.int32(0), i, jnp.int32(0))
        ),
        out_shape=jax.ShapeDtypeStruct((KH, ROWS2, LANES), jnp.int32),
    )(t32)


def _gather_mean(idx, w):
    mesh = plsc.VectorSubcoreMesh(core_axis_name="c", subcore_axis_name="s")

    @functools.partial(
        pl.kernel,
        out_type=jax.ShapeDtypeStruct((SEQ, WORDS, EMB), jnp.float32),
        mesh=mesh,
        compiler_params=pltpu.CompilerParams(use_tc_tiling_on_sc=False),
        scratch_types=[
            pltpu.VMEM((KH * TOK_BLK,), jnp.int32),
            pltpu.VMEM((KH * TOK_BLK, EMB), jnp.float32),
            pltpu.VMEM((SEQ_BLK, WORDS, EMB), jnp.float32),
            pltpu.SemaphoreType.DMA,
        ],
    )
    def k(idx_hbm, w_hbm, out_hbm, idx_v, rows_v, out_v, sem):
        wid = lax.axis_index("s") * 2 + lax.axis_index("c")

        def blk_body(b, carry):
            seq0 = wid * jnp.int32(SEQ_PER_W) + b * jnp.int32(SEQ_BLK)
            tok0 = seq0 * jnp.int32(WORDS)
            for r in range(KH):
                pltpu.sync_copy(
                    idx_hbm.at[pl.ds(jnp.int32(r * N_TOK) + tok0, TOK_BLK)],
                    idx_v.at[pl.ds(r * TOK_BLK, TOK_BLK)],
                )
            copies = [
                pltpu.async_copy(
                    w_hbm.at[idx_v.at[pl.ds(c * LANES, LANES)]],
                    rows_v.at[pl.ds(c * LANES, LANES)],
                    sem,
                )
                for c in range(IDX_CHUNKS)
            ]
            for cp in copies:
                cp.wait()

            def red_s(s, c2):
                def red_w(wd, c3):
                    j = s * jnp.int32(WORDS) + wd
                    for h in range(EMB // 16):
                        sl = pl.ds(h * 16, 16)
                        acc = (
                            rows_v[j, sl]
                            + rows_v[TOK_BLK + j, sl]
                            + rows_v[2 * TOK_BLK + j, sl]
                            + rows_v[3 * TOK_BLK + j, sl]
                        )
                        out_v[s, wd, sl] = acc * (1.0 / KH)
                    return c3

                return lax.fori_loop(jnp.int32(0), jnp.int32(WORDS), red_w, c2)

            lax.fori_loop(jnp.int32(0), jnp.int32(SEQ_BLK), red_s, 0)
            pltpu.sync_copy(out_v, out_hbm.at[pl.ds(seq0, SEQ_BLK), :, :])
            return carry

        lax.fori_loop(jnp.int32(0), jnp.int32(NBLK), blk_body, 0)

    return k(idx, w)


def kernel(t, W):
    exact_i64 = t.dtype == jnp.int64
    t32 = t.reshape(ROWS2, LANES).astype(jnp.int32)
    idx = _compute_idx(t32, exact_i64).reshape(-1)
    return _gather_mean(idx, W)


# speedup vs baseline: 1.7785x; 1.7785x over previous
import functools

import jax
import jax.numpy as jnp
from jax import lax
from jax.experimental import pallas as pl
from jax.experimental.pallas import tpu as pltpu
from jax.experimental.pallas import tpu_sc as plsc

NUM_ROWS = 100000
EMB = 32
KH = 4
HC = 73244475
HC0 = HC & 0xFFFF
HC1 = HC >> 16
M32 = 16824
M64C = 100000 - 51616

N_TOK = 16384 * 26
LANES = 128
ROWS2 = N_TOK // LANES
HB = 128
NW = 32
TOK_PER_W = N_TOK // NW
BLK = 128
NBLK = TOK_PER_W // BLK


def _hash_block_i64(t_ref, o_ref):
    x = t_ref[...].astype(jnp.uint32)
    outs = []
    for r in range(KH):
        lo = x + jnp.uint32(r)
        hi = jnp.zeros_like(lo)
        for _ in range(2):
            s_lo = ((lo >> 16) | (hi << 16)) ^ lo
            s_hi = (hi.astype(jnp.int32) >> 16).astype(jnp.uint32) ^ hi
            m_lo = s_lo * jnp.uint32(HC)
            x0 = s_lo & jnp.uint32(0xFFFF)
            x1 = s_lo >> 16
            m0 = x0 * jnp.uint32(HC0)
            mid = x1 * jnp.uint32(HC0) + x0 * jnp.uint32(HC1) + (m0 >> 16)
            mhi = x1 * jnp.uint32(HC1) + (mid >> 16)
            hi = s_hi * jnp.uint32(HC) + mhi
            lo = m_lo
        f_lo = ((lo >> 16) | (hi << 16)) ^ lo
        f_hi = (hi.astype(jnp.int32) >> 16).astype(jnp.uint32) ^ hi
        m = jnp.uint32(NUM_ROWS)
        p = ((f_hi % m) * jnp.uint32(M32)) % m
        q = (jnp.uint32(4) * p + f_lo % m) % m
        q = jnp.where(f_hi.astype(jnp.int32) < 0, (q + jnp.uint32(M64C)) % m, q)
        outs.append(q.astype(jnp.int32))
    o_ref[...] = jnp.stack(outs)


def _hash_block_i32(t_ref, o_ref):
    outs = []
    for r in range(KH):
        t = t_ref[...] + r
        t = ((t >> 16) ^ t) * HC
        t = ((t >> 16) ^ t) * HC
        t = (t >> 16) ^ t
        outs.append(jnp.mod(t, NUM_ROWS))
    o_ref[...] = jnp.stack(outs)


def _compute_idx(t32, exact_i64):
    body = _hash_block_i64 if exact_i64 else _hash_block_i32
    return pl.pallas_call(
        body,
        grid=(ROWS2 // HB,),
        in_specs=[pl.BlockSpec((HB, LANES), lambda i: (i, jnp.int32(0)))],
        out_specs=pl.BlockSpec(
            (KH, HB, LANES), lambda i: (jnp.int32(0), i, jnp.int32(0))
        ),
        out_shape=jax.ShapeDtypeStruct((KH, ROWS2, LANES), jnp.int32),
    )(t32)


def _gather_mean(idx, w):
    mesh = plsc.VectorSubcoreMesh(core_axis_name="c", subcore_axis_name="s")

    @functools.partial(
        pl.kernel,
        out_type=jax.ShapeDtypeStruct((N_TOK, EMB), jnp.float32),
        mesh=mesh,
        compiler_params=pltpu.CompilerParams(use_tc_tiling_on_sc=False),
        scratch_types=[
            pltpu.VMEM((KH * TOK_PER_W,), jnp.int32),
            pltpu.VMEM((KH * BLK, EMB), jnp.float32),
            pltpu.VMEM((KH * BLK, EMB), jnp.float32),
            pltpu.VMEM((BLK, EMB), jnp.float32),
            pltpu.VMEM((BLK, EMB), jnp.float32),
            pltpu.SemaphoreType.DMA,
            pltpu.SemaphoreType.DMA,
            pltpu.SemaphoreType.DMA,
            pltpu.SemaphoreType.DMA,
        ],
    )
    def k(idx_hbm, w_hbm, out_hbm, idx_all, rows0, rows1, out0, out1,
          gsem0, gsem1, osem0, osem1):
        wid = lax.axis_index("s") * 2 + lax.axis_index("c")
        tbase = wid * jnp.int32(TOK_PER_W)
        rows = (rows0, rows1)
        outs = (out0, out1)
        gsems = (gsem0, gsem1)
        osems = (osem0, osem1)

        for r in range(KH):
            pltpu.sync_copy(
                idx_hbm.at[pl.ds(jnp.int32(r * N_TOK) + tbase, TOK_PER_W)],
                idx_all.at[pl.ds(r * TOK_PER_W, TOK_PER_W)],
            )

        def fire(b, p):
            for r in range(KH):
                off = jnp.int32(r * TOK_PER_W) + b * jnp.int32(BLK)
                pltpu.async_copy(
                    w_hbm.at[idx_all.at[pl.ds(off, BLK)]],
                    rows[p].at[pl.ds(r * BLK, BLK)],
                    gsems[p],
                )

        def drain_gather(p):
            pltpu.make_async_copy(
                w_hbm.at[idx_all.at[pl.ds(jnp.int32(0), KH * BLK)]],
                rows[p],
                gsems[p],
            ).wait()

        def wait_out(p):
            pltpu.make_async_copy(
                outs[p],
                out_hbm.at[pl.ds(jnp.int32(0), BLK)],
                osems[p],
            ).wait()

        def step(b, p, do_fire, do_wait_out):
            if do_fire:
                fire(b + 1, 1 - p)
            if do_wait_out:
                wait_out(p)
            drain_gather(p)

            def red(i, c2):
                for h in range(EMB // 16):
                    sl = pl.ds(h * 16, 16)
                    acc = (
                        rows[p][i, sl]
                        + rows[p][BLK + i, sl]
                        + rows[p][2 * BLK + i, sl]
                        + rows[p][3 * BLK + i, sl]
                    )
                    outs[p][i, sl] = acc * (1.0 / KH)
                return c2

            lax.fori_loop(jnp.int32(0), jnp.int32(BLK), red, 0)
            pltpu.async_copy(
                outs[p],
                out_hbm.at[pl.ds(tbase + b * jnp.int32(BLK), BLK)],
                osems[p],
            )

        fire(jnp.int32(0), 0)
        step(jnp.int32(0), 0, True, False)
        step(jnp.int32(1), 1, True, False)

        def pair(g, carry):
            b = g * jnp.int32(2)
            step(b, 0, True, True)
            step(b + 1, 1, True, True)
            return carry

        lax.fori_loop(jnp.int32(1), jnp.int32(NBLK // 2 - 1), pair, 0)
        step(jnp.int32(NBLK - 2), 0, True, True)
        step(jnp.int32(NBLK - 1), 1, False, True)
        wait_out(0)
        wait_out(1)

    return k(idx, w)


def kernel(t, W):
    exact_i64 = t.dtype == jnp.int64
    t32 = t.reshape(ROWS2, LANES).astype(jnp.int32)
    idx = _compute_idx(t32, exact_i64).reshape(-1)
    return _gather_mean(idx, W)
